# Initial kernel scaffold; baseline (speedup 1.0000x reference)
import functools

import jax
import jax.numpy as jnp
from jax import lax
from jax.experimental import pallas as pl
from jax.experimental.pallas import tpu as pltpu
from jax.experimental.pallas import tpu_sc as plsc

N = 10000
E = 320000
D = 128

NC = 2
NS = 16
NW = NC * NS
CHUNK = 128
CPW = 80
EPW = CHUNK * CPW
E_PAD = NW * EPW
NP = 10240
RPT = NP // NS


def _deg_body(col_hbm, deg_hbm, col_vm, ones_vm, zer_vm, deg_sh):
    cid = lax.axis_index("c")
    sid = lax.axis_index("s")
    wid = sid * NC + cid

    for i in range(CHUNK // 16):
        ones_vm[pl.ds(i * 16, 16)] = jnp.ones((16,), jnp.float32)
        zer_vm[pl.ds(i * 16, 16)] = jnp.zeros((16,), jnp.float32)

    for i in range(RPT // CHUNK):
        pltpu.sync_copy(zer_vm, deg_sh.at[pl.ds(sid * RPT + i * CHUNK, CHUNK)])

    pltpu.sync_copy(col_hbm.at[wid], col_vm)
    plsc.subcore_barrier()

    def body(j, carry):
        pltpu.sync_copy(ones_vm, deg_sh.at[col_vm.at[j]], add=True)
        return carry

    lax.fori_loop(0, CPW, body, 0)
    plsc.subcore_barrier()
    pltpu.sync_copy(deg_sh.at[pl.ds(sid * RPT, RPT)],
                    deg_hbm.at[pl.ds(cid * NP + sid * RPT, RPT)])


def _gather_scatter_body(y_hbm, row_hbm, col_hbm, zer_hbm, g_hbm,
                         row_vm, col_vm, buf0, buf1, sem0, sem1, g_sh):
    cid = lax.axis_index("c")
    sid = lax.axis_index("s")
    wid = sid * NC + cid

    pltpu.sync_copy(row_hbm.at[wid], row_vm)
    pltpu.sync_copy(col_hbm.at[wid], col_vm)
    pltpu.sync_copy(zer_hbm.at[pl.ds(sid * RPT, RPT)],
                    g_sh.at[pl.ds(sid * RPT, RPT)])
    plsc.subcore_barrier()

    pltpu.async_copy(y_hbm.at[row_vm.at[0]], buf0, sem0)

    def body(p, carry):
        c0 = 2 * p
        pltpu.async_copy(y_hbm.at[row_vm.at[c0 + 1]], buf1, sem1)
        pltpu.make_async_copy(y_hbm.at[row_vm.at[c0]], buf0, sem0).wait()
        pltpu.sync_copy(buf0, g_sh.at[col_vm.at[c0]], add=True)

        @pl.when(p < CPW // 2 - 1)
        def _():
            pltpu.async_copy(y_hbm.at[row_vm.at[c0 + 2]], buf0, sem0)

        pltpu.make_async_copy(y_hbm.at[row_vm.at[c0 + 1]], buf1, sem1).wait()
        pltpu.sync_copy(buf1, g_sh.at[col_vm.at[c0 + 1]], add=True)
        return carry

    lax.fori_loop(0, CPW // 2, body, 0)
    plsc.subcore_barrier()
    pltpu.sync_copy(g_sh.at[pl.ds(sid * RPT, RPT)],
                    g_hbm.at[pl.ds(cid * NP + sid * RPT, RPT)])


def _y_body(x_ref, d0_ref, d1_ref, y_ref):
    deg = d0_ref[...] + d1_ref[...] + 1.0
    y_ref[...] = x_ref[...] * lax.rsqrt(deg)


def _out_body(x_ref, d0_ref, d1_ref, g0_ref, g1_ref, cw_ref, lw_ref,
              cb_ref, lb_ref, o_ref):
    deg = d0_ref[...] + d1_ref[...] + 1.0
    s = (g0_ref[...] + g1_ref[...]) * lax.rsqrt(deg) + x_ref[...] / deg
    w = jnp.dot(cw_ref[...], lw_ref[...], preferred_element_type=jnp.float32)
    b = jnp.dot(cb_ref[...], lw_ref[...], preferred_element_type=jnp.float32)
    o_ref[...] = jnp.dot(s, w, preferred_element_type=jnp.float32) + b + lb_ref[...]


@jax.jit
def kernel(x, edge_index, conv_W, conv_b, lin_W, lin_b):
    row = jnp.pad(edge_index[0], (0, E_PAD - E)).reshape(NW, CPW, CHUNK)
    col = jnp.pad(edge_index[1], (0, E_PAD - E),
                  constant_values=N).reshape(NW, CPW, CHUNK)
    x_pad = jnp.pad(x, ((0, NP - N), (0, 0)))
    zer = jnp.zeros((NP, D), jnp.float32)

    deg_kernel = pl.kernel(
        _deg_body,
        out_type=jax.ShapeDtypeStruct((NC * NP,), jnp.float32),
        mesh=plsc.VectorSubcoreMesh(core_axis_name="c", subcore_axis_name="s"),
        scratch_types=[
            pltpu.VMEM((CPW, CHUNK), jnp.int32),
            pltpu.VMEM((CHUNK,), jnp.float32),
            pltpu.VMEM((CHUNK,), jnp.float32),
            pltpu.VMEM_SHARED((NP,), jnp.float32),
        ],
    )
    deg2 = deg_kernel(col)
    d0 = deg2[:NP].reshape(NP, 1)
    d1 = deg2[NP:].reshape(NP, 1)

    blk = 1024
    grid = NP // blk
    y = pl.pallas_call(
        _y_body,
        grid=(grid,),
        in_specs=[
            pl.BlockSpec((blk, D), lambda i: (i, 0)),
            pl.BlockSpec((blk, 1), lambda i: (i, 0)),
            pl.BlockSpec((blk, 1), lambda i: (i, 0)),
        ],
        out_specs=pl.BlockSpec((blk, D), lambda i: (i, 0)),
        out_shape=jax.ShapeDtypeStruct((NP, D), jnp.float32),
    )(x_pad, d0, d1)

    gs_kernel = pl.kernel(
        _gather_scatter_body,
        out_type=jax.ShapeDtypeStruct((NC * NP, D), jnp.float32),
        mesh=plsc.VectorSubcoreMesh(core_axis_name="c", subcore_axis_name="s"),
        scratch_types=[
            pltpu.VMEM((CPW, CHUNK), jnp.int32),
            pltpu.VMEM((CPW, CHUNK), jnp.int32),
            pltpu.VMEM((CHUNK, D), jnp.float32),
            pltpu.VMEM((CHUNK, D), jnp.float32),
            pltpu.SemaphoreType.DMA,
            pltpu.SemaphoreType.DMA,
            pltpu.VMEM_SHARED((NP, D), jnp.float32),
        ],
    )
    g2 = gs_kernel(y, row, col, zer)

    out = pl.pallas_call(
        _out_body,
        grid=(grid,),
        in_specs=[
            pl.BlockSpec((blk, D), lambda i: (i, 0)),
            pl.BlockSpec((blk, 1), lambda i: (i, 0)),
            pl.BlockSpec((blk, 1), lambda i: (i, 0)),
            pl.BlockSpec((blk, D), lambda i: (i, 0)),
            pl.BlockSpec((blk, D), lambda i: (i, 0)),
            pl.BlockSpec((D, D), lambda i: (0, 0)),
            pl.BlockSpec((D, D), lambda i: (0, 0)),
            pl.BlockSpec((1, D), lambda i: (0, 0)),
            pl.BlockSpec((1, D), lambda i: (0, 0)),
        ],
        out_specs=pl.BlockSpec((blk, D), lambda i: (i, 0)),
        out_shape=jax.ShapeDtypeStruct((NP, D), jnp.float32),
    )(x_pad, d0, d1, g2[:NP], g2[NP:], conv_W, lin_W,
      conv_b.reshape(1, D), lin_b.reshape(1, D))

    return out[:N]


# speedup vs baseline: 12.6420x; 12.6420x over previous
import functools

import jax
import jax.numpy as jnp
from jax import lax
from jax.experimental import pallas as pl
from jax.experimental.pallas import tpu as pltpu
from jax.experimental.pallas import tpu_sc as plsc

N = 10000
E = 320000
D = 128

NC = 2
NS = 16
NW = NC * NS
CHUNK = 128
CPW = 80
EPW = CHUNK * CPW
E_PAD = NW * EPW
NP = 10240
RPT = NP // NS


def _deg_body(col_hbm, deg_hbm, deg_sh):
    cid = lax.axis_index("c")
    sid = lax.axis_index("s")
    wid = sid * NC + cid

    def scoped(col_vm, ones_vm, zer_vm):
        for i in range(CHUNK // 16):
            ones_vm[pl.ds(i * 16, 16)] = jnp.ones((16,), jnp.float32)
            zer_vm[pl.ds(i * 16, 16)] = jnp.zeros((16,), jnp.float32)

        for i in range(RPT // CHUNK):
            pltpu.sync_copy(zer_vm,
                            deg_sh.at[pl.ds(sid * RPT + i * CHUNK, CHUNK)])

        pltpu.sync_copy(col_hbm.at[wid], col_vm)
        plsc.subcore_barrier()

        def body(j, carry):
            pltpu.sync_copy(ones_vm, deg_sh.at[col_vm.at[j]], add=True)
            return carry

        lax.fori_loop(0, CPW, body, 0)
        plsc.subcore_barrier()
        pltpu.sync_copy(deg_sh.at[pl.ds(sid * RPT, RPT)],
                        deg_hbm.at[pl.ds(cid * NP + sid * RPT, RPT)])

    pl.run_scoped(
        scoped,
        pltpu.VMEM((CPW, CHUNK), jnp.int32),
        pltpu.VMEM((CHUNK,), jnp.float32),
        pltpu.VMEM((CHUNK,), jnp.float32),
    )


def _gather_scatter_body(y_hbm, row_hbm, col_hbm, zer_hbm, g_hbm, g_sh):
    cid = lax.axis_index("c")
    sid = lax.axis_index("s")
    wid = sid * NC + cid

    def scoped(row_vm, col_vm, buf, sem):
        pltpu.sync_copy(row_hbm.at[wid], row_vm)
        pltpu.sync_copy(col_hbm.at[wid], col_vm)

        pltpu.sync_copy(zer_hbm.at[pl.ds(0, CHUNK)], buf)

        def zbody(i, carry):
            pltpu.sync_copy(buf, g_sh.at[pl.ds(sid * RPT + i * CHUNK, CHUNK)])
            return carry

        lax.fori_loop(0, RPT // CHUNK, zbody, 0)
        plsc.subcore_barrier()

        def body(c, carry):
            pltpu.async_copy(y_hbm.at[row_vm.at[c]], buf, sem).wait()
            pltpu.sync_copy(buf, g_sh.at[col_vm.at[c]], add=True)
            return carry

        lax.fori_loop(0, CPW, body, 0)
        plsc.subcore_barrier()

        def obody(i, carry):
            pltpu.sync_copy(g_sh.at[pl.ds(sid * RPT + i * CHUNK, CHUNK)], buf)
            pltpu.sync_copy(
                buf, g_hbm.at[pl.ds(cid * NP + sid * RPT + i * CHUNK, CHUNK)])
            return carry

        lax.fori_loop(0, RPT // CHUNK, obody, 0)

    pl.run_scoped(
        scoped,
        pltpu.VMEM((CPW, CHUNK), jnp.int32),
        pltpu.VMEM((CPW, CHUNK), jnp.int32),
        pltpu.VMEM((CHUNK, D), jnp.float32),
        pltpu.SemaphoreType.DMA,
    )


def _y_body(x_ref, d0_ref, d1_ref, y_ref):
    deg = d0_ref[...] + d1_ref[...] + 1.0
    y_ref[...] = x_ref[...] * lax.rsqrt(deg)


def _out_body(x_ref, d0_ref, d1_ref, g0_ref, g1_ref, cw_ref, lw_ref,
              cb_ref, lb_ref, o_ref):
    deg = d0_ref[...] + d1_ref[...] + 1.0
    s = (g0_ref[...] + g1_ref[...]) * lax.rsqrt(deg) + x_ref[...] / deg
    w = jnp.dot(cw_ref[...], lw_ref[...], preferred_element_type=jnp.float32)
    b = jnp.dot(cb_ref[...], lw_ref[...], preferred_element_type=jnp.float32)
    o_ref[...] = jnp.dot(s, w, preferred_element_type=jnp.float32) + b + lb_ref[...]


@jax.jit
def kernel(x, edge_index, conv_W, conv_b, lin_W, lin_b):
    row = jnp.pad(edge_index[0], (0, E_PAD - E)).reshape(NW, CPW, CHUNK)
    col = jnp.pad(edge_index[1], (0, E_PAD - E),
                  constant_values=N).reshape(NW, CPW, CHUNK)
    x_pad = jnp.pad(x, ((0, NP - N), (0, 0)))
    zer = jnp.zeros((NP, D), jnp.float32)

    deg_kernel = pl.kernel(
        _deg_body,
        out_type=jax.ShapeDtypeStruct((NC * NP,), jnp.float32),
        mesh=plsc.VectorSubcoreMesh(core_axis_name="c", subcore_axis_name="s"),
        scratch_types=[
            pltpu.VMEM_SHARED((NP,), jnp.float32),
        ],
    )
    deg2 = deg_kernel(col)
    d0 = deg2[:NP].reshape(NP, 1)
    d1 = deg2[NP:].reshape(NP, 1)

    blk = 1024
    grid = NP // blk
    y = pl.pallas_call(
        _y_body,
        grid=(grid,),
        in_specs=[
            pl.BlockSpec((blk, D), lambda i: (i, 0)),
            pl.BlockSpec((blk, 1), lambda i: (i, 0)),
            pl.BlockSpec((blk, 1), lambda i: (i, 0)),
        ],
        out_specs=pl.BlockSpec((blk, D), lambda i: (i, 0)),
        out_shape=jax.ShapeDtypeStruct((NP, D), jnp.float32),
    )(x_pad, d0, d1)

    gs_kernel = pl.kernel(
        _gather_scatter_body,
        out_type=jax.ShapeDtypeStruct((NC * NP, D), jnp.float32),
        mesh=plsc.VectorSubcoreMesh(core_axis_name="c", subcore_axis_name="s"),
        scratch_types=[
            pltpu.VMEM_SHARED((NP, D), jnp.float32),
        ],
    )
    g2 = gs_kernel(y, row, col, zer)

    out = pl.pallas_call(
        _out_body,
        grid=(grid,),
        in_specs=[
            pl.BlockSpec((blk, D), lambda i: (i, 0)),
            pl.BlockSpec((blk, 1), lambda i: (i, 0)),
            pl.BlockSpec((blk, 1), lambda i: (i, 0)),
            pl.BlockSpec((blk, D), lambda i: (i, 0)),
            pl.BlockSpec((blk, D), lambda i: (i, 0)),
            pl.BlockSpec((D, D), lambda i: (0, 0)),
            pl.BlockSpec((D, D), lambda i: (0, 0)),
            pl.BlockSpec((1, D), lambda i: (0, 0)),
            pl.BlockSpec((1, D), lambda i: (0, 0)),
        ],
        out_specs=pl.BlockSpec((blk, D), lambda i: (i, 0)),
        out_shape=jax.ShapeDtypeStruct((NP, D), jnp.float32),
    )(x_pad, d0, d1, g2[:NP], g2[NP:], conv_W, lin_W,
      conv_b.reshape(1, D), lin_b.reshape(1, D))

    return out[:N]


# speedup vs baseline: 14.1292x; 1.1176x over previous
import functools

import jax
import jax.numpy as jnp
from jax import lax
from jax.experimental import pallas as pl
from jax.experimental.pallas import tpu as pltpu
from jax.experimental.pallas import tpu_sc as plsc

N = 10000
E = 320000
D = 128

NC = 2
NS = 16
NW = NC * NS
CHUNK = 128
CPW = 80
EPW = CHUNK * CPW
E_PAD = NW * EPW
NP = 10240
RPT = NP // NS


def _deg_body(col_hbm, deg_hbm, deg_sh):
    cid = lax.axis_index("c")
    sid = lax.axis_index("s")
    wid = sid * NC + cid

    def scoped(col_vm, ones_vm, zer_vm):
        for i in range(CHUNK // 16):
            ones_vm[pl.ds(i * 16, 16)] = jnp.ones((16,), jnp.float32)
            zer_vm[pl.ds(i * 16, 16)] = jnp.zeros((16,), jnp.float32)

        for i in range(RPT // CHUNK):
            pltpu.sync_copy(zer_vm,
                            deg_sh.at[pl.ds(sid * RPT + i * CHUNK, CHUNK)])

        pltpu.sync_copy(col_hbm.at[wid], col_vm)
        plsc.subcore_barrier()

        def body(j, carry):
            pltpu.sync_copy(ones_vm, deg_sh.at[col_vm.at[j]], add=True)
            return carry

        lax.fori_loop(0, CPW, body, 0)
        plsc.subcore_barrier()
        pltpu.sync_copy(deg_sh.at[pl.ds(sid * RPT, RPT)],
                        deg_hbm.at[pl.ds(cid * NP + sid * RPT, RPT)])

    pl.run_scoped(
        scoped,
        pltpu.VMEM((CPW, CHUNK), jnp.int32),
        pltpu.VMEM((CHUNK,), jnp.float32),
        pltpu.VMEM((CHUNK,), jnp.float32),
    )


def _gather_scatter_body(y_hbm, row_hbm, col_hbm, zer_hbm, g_hbm, g_sh):
    cid = lax.axis_index("c")
    sid = lax.axis_index("s")
    wid = sid * NC + cid

    def scoped(row_vm, col_vm, buf, sem):
        pltpu.sync_copy(row_hbm.at[wid], row_vm)
        pltpu.sync_copy(col_hbm.at[wid], col_vm)

        pltpu.sync_copy(zer_hbm.at[pl.ds(0, CHUNK)], buf)

        def zbody(i, carry):
            pltpu.sync_copy(buf, g_sh.at[pl.ds(sid * RPT + i * CHUNK, CHUNK)])
            return carry

        lax.fori_loop(0, RPT // CHUNK, zbody, 0)
        plsc.subcore_barrier()

        def body(c, carry):
            pltpu.async_copy(y_hbm.at[row_vm.at[c]], buf, sem).wait()
            pltpu.sync_copy(buf, g_sh.at[col_vm.at[c]], add=True)
            return carry

        lax.fori_loop(0, CPW, body, 0)
        plsc.subcore_barrier()

        def obody(i, carry):
            pltpu.sync_copy(g_sh.at[pl.ds(sid * RPT + i * CHUNK, CHUNK)], buf)
            pltpu.sync_copy(
                buf, g_hbm.at[pl.ds(cid * NP + sid * RPT + i * CHUNK, CHUNK)])
            return carry

        lax.fori_loop(0, RPT // CHUNK, obody, 0)

    pl.run_scoped(
        scoped,
        pltpu.VMEM((CPW, CHUNK), jnp.int32),
        pltpu.VMEM((CPW, CHUNK), jnp.int32),
        pltpu.VMEM((CHUNK, D), jnp.float32),
        pltpu.SemaphoreType.DMA,
    )


def _y_body(x_ref, d0_ref, d1_ref, y_ref):
    deg = d0_ref[...] + d1_ref[...] + 1.0
    y_ref[...] = x_ref[...] * lax.rsqrt(deg)


def _out_body(x_ref, d0_ref, d1_ref, g0_ref, g1_ref, cw_ref, lw_ref,
              cb_ref, lb_ref, o_ref):
    deg = d0_ref[...] + d1_ref[...] + 1.0
    s = (g0_ref[...] + g1_ref[...]) * lax.rsqrt(deg) + x_ref[...] / deg
    w = jnp.dot(cw_ref[...], lw_ref[...], preferred_element_type=jnp.float32)
    b = jnp.dot(cb_ref[...], lw_ref[...], preferred_element_type=jnp.float32)
    o_ref[...] = jnp.dot(s, w, preferred_element_type=jnp.float32) + b + lb_ref[...]


@jax.jit
def kernel(x, edge_index, conv_W, conv_b, lin_W, lin_b):
    ppw = EPW - E // NW
    pad_r = jnp.zeros((NW, ppw), jnp.int32)
    pad_c = jnp.broadcast_to(N + jnp.arange(ppw, dtype=jnp.int32) % (NP - N),
                             (NW, ppw))
    row = jnp.concatenate([edge_index[0].reshape(NW, E // NW), pad_r],
                          axis=1).reshape(NW, CPW, CHUNK)
    col = jnp.concatenate([edge_index[1].reshape(NW, E // NW), pad_c],
                          axis=1).reshape(NW, CPW, CHUNK)
    x_pad = jnp.pad(x, ((0, NP - N), (0, 0)))
    zer = jnp.zeros((NP, D), jnp.float32)

    deg_kernel = pl.kernel(
        _deg_body,
        out_type=jax.ShapeDtypeStruct((NC * NP,), jnp.float32),
        mesh=plsc.VectorSubcoreMesh(core_axis_name="c", subcore_axis_name="s"),
        scratch_types=[
            pltpu.VMEM_SHARED((NP,), jnp.float32),
        ],
    )
    deg2 = deg_kernel(col)
    d0 = deg2[:NP].reshape(NP, 1)
    d1 = deg2[NP:].reshape(NP, 1)

    blk = 1024
    grid = NP // blk
    y = pl.pallas_call(
        _y_body,
        grid=(grid,),
        in_specs=[
            pl.BlockSpec((blk, D), lambda i: (i, 0)),
            pl.BlockSpec((blk, 1), lambda i: (i, 0)),
            pl.BlockSpec((blk, 1), lambda i: (i, 0)),
        ],
        out_specs=pl.BlockSpec((blk, D), lambda i: (i, 0)),
        out_shape=jax.ShapeDtypeStruct((NP, D), jnp.float32),
    )(x_pad, d0, d1)

    gs_kernel = pl.kernel(
        _gather_scatter_body,
        out_type=jax.ShapeDtypeStruct((NC * NP, D), jnp.float32),
        mesh=plsc.VectorSubcoreMesh(core_axis_name="c", subcore_axis_name="s"),
        scratch_types=[
            pltpu.VMEM_SHARED((NP, D), jnp.float32),
        ],
    )
    g2 = gs_kernel(y, row, col, zer)

    out = pl.pallas_call(
        _out_body,
        grid=(grid,),
        in_specs=[
            pl.BlockSpec((blk, D), lambda i: (i, 0)),
            pl.BlockSpec((blk, 1), lambda i: (i, 0)),
            pl.BlockSpec((blk, 1), lambda i: (i, 0)),
            pl.BlockSpec((blk, D), lambda i: (i, 0)),
            pl.BlockSpec((blk, D), lambda i: (i, 0)),
            pl.BlockSpec((D, D), lambda i: (0, 0)),
            pl.BlockSpec((D, D), lambda i: (0, 0)),
            pl.BlockSpec((1, D), lambda i: (0, 0)),
            pl.BlockSpec((1, D), lambda i: (0, 0)),
        ],
        out_specs=pl.BlockSpec((blk, D), lambda i: (i, 0)),
        out_shape=jax.ShapeDtypeStruct((NP, D), jnp.float32),
    )(x_pad, d0, d1, g2[:NP], g2[NP:], conv_W, lin_W,
      conv_b.reshape(1, D), lin_b.reshape(1, D))

    return out[:N]


# speedup vs baseline: 14.6013x; 1.0334x over previous
import functools

import jax
import jax.numpy as jnp
from jax import lax
from jax.experimental import pallas as pl
from jax.experimental.pallas import tpu as pltpu
from jax.experimental.pallas import tpu_sc as plsc

N = 10000
E = 320000
D = 128

NC = 2
NS = 16
NW = NC * NS
CHUNK = 128
CPW = 80
EPW = CHUNK * CPW
E_PAD = NW * EPW
NP = 10240
RPT = NP // NS
GB = 32
IG = 16


def _deg_body(col_hbm, deg_hbm, deg_sh):
    cid = lax.axis_index("c")
    sid = lax.axis_index("s")
    wid = sid * NC + cid

    def scoped(col_vm, ones_vm, zer_vm):
        for i in range(CHUNK // 16):
            ones_vm[pl.ds(i * 16, 16)] = jnp.ones((16,), jnp.float32)
            zer_vm[pl.ds(i * 16, 16)] = jnp.zeros((16,), jnp.float32)

        for i in range(RPT // CHUNK):
            pltpu.sync_copy(zer_vm,
                            deg_sh.at[pl.ds(sid * RPT + i * CHUNK, CHUNK)])

        pltpu.sync_copy(col_hbm.at[wid], col_vm)
        plsc.subcore_barrier()

        def body(j, carry):
            pltpu.sync_copy(ones_vm, deg_sh.at[col_vm.at[j]], add=True)
            return carry

        lax.fori_loop(0, CPW, body, 0)
        plsc.subcore_barrier()
        pltpu.sync_copy(deg_sh.at[pl.ds(sid * RPT, RPT)],
                        deg_hbm.at[pl.ds(cid * NP + sid * RPT, RPT)])

    pl.run_scoped(
        scoped,
        pltpu.VMEM((CPW, CHUNK), jnp.int32),
        pltpu.VMEM((CHUNK,), jnp.float32),
        pltpu.VMEM((CHUNK,), jnp.float32),
    )


def _gather_scatter_body(y_hbm, row_hbm, col_hbm, zer_hbm, g_hbm, g_sh):
    cid = lax.axis_index("c")
    sid = lax.axis_index("s")
    wid = sid * NC + cid

    def scoped(row_vm, col_vm, gbuf, bufa, bufb, sema, semb):
        pltpu.sync_copy(zer_hbm.at[pl.ds(0, GB)], gbuf)

        def zbody(i, carry):
            pltpu.sync_copy(gbuf, g_sh.at[pl.ds(sid * RPT + i * GB, GB)])
            return carry

        lax.fori_loop(0, RPT // GB, zbody, 0)
        plsc.subcore_barrier()

        def group(gi, carry):
            pltpu.sync_copy(row_hbm.at[wid, pl.ds(gi * IG, IG)], row_vm)
            pltpu.sync_copy(col_hbm.at[wid, pl.ds(gi * IG, IG)], col_vm)

            def pair(p, ic):
                c0 = 2 * p
                cpa = pltpu.async_copy(y_hbm.at[row_vm.at[c0]], bufa, sema)
                cpb = pltpu.async_copy(y_hbm.at[row_vm.at[c0 + 1]], bufb,
                                       semb)
                cpa.wait()
                pltpu.sync_copy(bufa, g_sh.at[col_vm.at[c0]], add=True)
                cpb.wait()
                pltpu.sync_copy(bufb, g_sh.at[col_vm.at[c0 + 1]], add=True)
                return ic

            lax.fori_loop(0, IG // 2, pair, 0)
            return carry

        lax.fori_loop(0, CPW // IG, group, 0)
        plsc.subcore_barrier()

        def obody(i, carry):
            pltpu.sync_copy(g_sh.at[pl.ds(sid * RPT + i * GB, GB)], gbuf)
            pltpu.sync_copy(
                gbuf, g_hbm.at[pl.ds(cid * NP + sid * RPT + i * GB, GB)])
            return carry

        lax.fori_loop(0, RPT // GB, obody, 0)

    pl.run_scoped(
        scoped,
        pltpu.VMEM((IG, CHUNK), jnp.int32),
        pltpu.VMEM((IG, CHUNK), jnp.int32),
        pltpu.VMEM((GB, D), jnp.float32),
        pltpu.VMEM((CHUNK, D), jnp.float32),
        pltpu.VMEM((CHUNK, D), jnp.float32),
        pltpu.SemaphoreType.DMA,
        pltpu.SemaphoreType.DMA,
    )


def _y_body(x_ref, d0_ref, d1_ref, y_ref):
    deg = d0_ref[...] + d1_ref[...] + 1.0
    y_ref[...] = x_ref[...] * lax.rsqrt(deg)


def _out_body(x_ref, d0_ref, d1_ref, g0_ref, g1_ref, cw_ref, lw_ref,
              cb_ref, lb_ref, o_ref):
    deg = d0_ref[...] + d1_ref[...] + 1.0
    s = (g0_ref[...] + g1_ref[...]) * lax.rsqrt(deg) + x_ref[...] / deg
    w = jnp.dot(cw_ref[...], lw_ref[...], preferred_element_type=jnp.float32)
    b = jnp.dot(cb_ref[...], lw_ref[...], preferred_element_type=jnp.float32)
    o_ref[...] = jnp.dot(s, w, preferred_element_type=jnp.float32) + b + lb_ref[...]


@jax.jit
def kernel(x, edge_index, conv_W, conv_b, lin_W, lin_b):
    ppw = EPW - E // NW
    pad_r = jnp.zeros((NW, ppw), jnp.int32)
    pad_c = jnp.broadcast_to(N + jnp.arange(ppw, dtype=jnp.int32) % (NP - N),
                             (NW, ppw))
    row = jnp.concatenate([edge_index[0].reshape(NW, E // NW), pad_r],
                          axis=1).reshape(NW, CPW, CHUNK)
    col = jnp.concatenate([edge_index[1].reshape(NW, E // NW), pad_c],
                          axis=1).reshape(NW, CPW, CHUNK)
    x_pad = jnp.pad(x, ((0, NP - N), (0, 0)))
    zer = jnp.zeros((NP, D), jnp.float32)

    deg_kernel = pl.kernel(
        _deg_body,
        out_type=jax.ShapeDtypeStruct((NC * NP,), jnp.float32),
        mesh=plsc.VectorSubcoreMesh(core_axis_name="c", subcore_axis_name="s"),
        scratch_types=[
            pltpu.VMEM_SHARED((NP,), jnp.float32),
        ],
    )
    deg2 = deg_kernel(col)
    d0 = deg2[:NP].reshape(NP, 1)
    d1 = deg2[NP:].reshape(NP, 1)

    blk = 1024
    grid = NP // blk
    y = pl.pallas_call(
        _y_body,
        grid=(grid,),
        in_specs=[
            pl.BlockSpec((blk, D), lambda i: (i, 0)),
            pl.BlockSpec((blk, 1), lambda i: (i, 0)),
            pl.BlockSpec((blk, 1), lambda i: (i, 0)),
        ],
        out_specs=pl.BlockSpec((blk, D), lambda i: (i, 0)),
        out_shape=jax.ShapeDtypeStruct((NP, D), jnp.float32),
    )(x_pad, d0, d1)

    gs_kernel = pl.kernel(
        _gather_scatter_body,
        out_type=jax.ShapeDtypeStruct((NC * NP, D), jnp.float32),
        mesh=plsc.VectorSubcoreMesh(core_axis_name="c", subcore_axis_name="s"),
        scratch_types=[
            pltpu.VMEM_SHARED((NP, D), jnp.float32),
        ],
    )
    g2 = gs_kernel(y, row, col, zer)

    out = pl.pallas_call(
        _out_body,
        grid=(grid,),
        in_specs=[
            pl.BlockSpec((blk, D), lambda i: (i, 0)),
            pl.BlockSpec((blk, 1), lambda i: (i, 0)),
            pl.BlockSpec((blk, 1), lambda i: (i, 0)),
            pl.BlockSpec((blk, D), lambda i: (i, 0)),
            pl.BlockSpec((blk, D), lambda i: (i, 0)),
            pl.BlockSpec((D, D), lambda i: (0, 0)),
            pl.BlockSpec((D, D), lambda i: (0, 0)),
            pl.BlockSpec((1, D), lambda i: (0, 0)),
            pl.BlockSpec((1, D), lambda i: (0, 0)),
        ],
        out_specs=pl.BlockSpec((blk, D), lambda i: (i, 0)),
        out_shape=jax.ShapeDtypeStruct((NP, D), jnp.float32),
    )(x_pad, d0, d1, g2[:NP], g2[NP:], conv_W, lin_W,
      conv_b.reshape(1, D), lin_b.reshape(1, D))

    return out[:N]
